# baseline (speedup 1.0000x reference)
import functools

import jax
import jax.numpy as jnp
from jax import lax
from jax.experimental import pallas as pl
from jax.experimental.pallas import tpu as pltpu
from jax.experimental.pallas import tpu_sc as plsc

_VOCAB = 100000
_EMBED = 64
_CTX = 20
_HID = 256
_BATCH = 1024

_CH = 2048
_NFULL = _VOCAB // _CH
_MAIN = _NFULL * _CH
_TAILW = _VOCAB - _MAIN
_NSLOT = 4
_HALF = _BATCH // 2


_IDX_CHUNK = 128


def _sc_gather(table, idx):
    info = plsc.get_sparse_core_info()
    nc, ns = info.num_cores, info.num_subcores
    nw = nc * ns
    n = idx.shape[0]
    assert n % (nw * _IDX_CHUNK) == 0
    per_w = n // nw
    chunks = per_w // _IDX_CHUNK
    idx3 = idx.reshape(nw, chunks, _IDX_CHUNK)
    mesh = plsc.VectorSubcoreMesh(core_axis_name="c", subcore_axis_name="s")

    @functools.partial(
        pl.kernel,
        mesh=mesh,
        out_type=jax.ShapeDtypeStruct((n, _EMBED), jnp.float32),
        scratch_types=[
            pltpu.VMEM((chunks, _IDX_CHUNK), jnp.int32),
            pltpu.VMEM((per_w, _EMBED), jnp.float32),
            pltpu.SemaphoreType.DMA,
        ],
        compiler_params=pltpu.CompilerParams(use_tc_tiling_on_sc=False),
    )
    def gather_k(table_hbm, idx_hbm, out_hbm, idx_v, rows_v, sem):
        wid = lax.axis_index("s") * nc + lax.axis_index("c")
        base = wid * per_w
        pltpu.sync_copy(idx_hbm.at[wid], idx_v)
        cps = []
        for i in range(chunks):
            cps.append(pltpu.async_copy(
                table_hbm.at[idx_v.at[i]],
                rows_v.at[pl.ds(i * _IDX_CHUNK, _IDX_CHUNK)],
                sem,
            ))
        for cp in cps:
            cp.wait()
        pltpu.sync_copy(rows_v, out_hbm.at[pl.ds(base, per_w)])

    return gather_k(table, idx3)


def _mlp1_body(x_ref, w1_ref, b1_ref, h_ref):
    acc = jnp.dot(x_ref[...], w1_ref[...],
                  preferred_element_type=jnp.float32,
                  precision=lax.Precision.HIGHEST)
    h_ref[...] = jnp.maximum(acc + b1_ref[...], 0.0)


def _mlp1(x, w1, b1r):
    return pl.pallas_call(
        _mlp1_body,
        out_shape=jax.ShapeDtypeStruct((_BATCH, _HID), jnp.float32),
    )(x, w1, b1r)


def _chunk_dmas(w2_hbm, big_hbm, w2_buf, out_buf, in_sem, out_sem):
    def in_cp(j, slot):
        return pltpu.make_async_copy(
            w2_hbm.at[:, pl.ds(pl.multiple_of(j * _CH, _CH), _CH)],
            w2_buf.at[slot],
            in_sem.at[slot])

    def out_cp(j, slot, half):
        return pltpu.make_async_copy(
            out_buf.at[slot, pl.ds(half * _HALF, _HALF)],
            big_hbm.at[pl.ds(half * _HALF, _HALF),
                       pl.ds(pl.multiple_of(j * _CH, _CH), _CH)],
            out_sem.at[slot, half])

    return in_cp, out_cp


def _pass1_body(h_ref, b2m_ref, w2_hbm, logits_hbm, m_ref, s_ref,
                w2_buf, out_buf, in_sem, out_sem):
    h = h_ref[...]
    in_cp, out_cp = _chunk_dmas(w2_hbm, logits_hbm, w2_buf, out_buf,
                                in_sem, out_sem)
    for r in range(_NSLOT):
        in_cp(r, r).start()

    def step(k, carry):
        m, s = carry
        for r in range(_NSLOT):
            j = k * _NSLOT + r
            in_cp(j, r).wait()
            tile = jnp.dot(h, w2_buf[r, :, :],
                           preferred_element_type=jnp.float32)
            tile = tile + b2m_ref[pl.ds(j, 1), :]

            @pl.when(k >= 1)
            def _():
                out_cp(j, r, 0).wait()
                out_cp(j, r, 1).wait()

            out_buf[r, :, :] = tile
            out_cp(j, r, 0).start()
            out_cp(j, r, 1).start()

            @pl.when(k < _NFULL // _NSLOT - 1)
            def _():
                in_cp(j + _NSLOT, r).start()

            tmax = jnp.max(tile, axis=1, keepdims=True)
            m_new = jnp.maximum(m, tmax)
            s = (s * jnp.exp(m - m_new)
                 + jnp.sum(jnp.exp(tile - m_new), axis=1, keepdims=True))
            m = m_new
        return m, s

    m0 = jnp.full((_BATCH, 1), -jnp.inf, jnp.float32)
    s0 = jnp.zeros((_BATCH, 1), jnp.float32)
    m, s = lax.fori_loop(0, _NFULL // _NSLOT, step, (m0, s0))
    m_ref[...] = m
    s_ref[...] = s
    for r in range(_NSLOT):
        out_cp(_NFULL - _NSLOT + r, r, 0).wait()
        out_cp(_NFULL - _NSLOT + r, r, 1).wait()


def _pass1(h, b2m, w2):
    return pl.pallas_call(
        _pass1_body,
        in_specs=[
            pl.BlockSpec(memory_space=pltpu.MemorySpace.VMEM),
            pl.BlockSpec(memory_space=pltpu.MemorySpace.VMEM),
            pl.BlockSpec(memory_space=pltpu.MemorySpace.HBM),
        ],
        out_specs=[
            pl.BlockSpec(memory_space=pltpu.MemorySpace.HBM),
            pl.BlockSpec(memory_space=pltpu.MemorySpace.VMEM),
            pl.BlockSpec(memory_space=pltpu.MemorySpace.VMEM),
        ],
        out_shape=[
            jax.ShapeDtypeStruct((_BATCH, _VOCAB), jnp.float32),
            jax.ShapeDtypeStruct((_BATCH, 1), jnp.float32),
            jax.ShapeDtypeStruct((_BATCH, 1), jnp.float32),
        ],
        scratch_shapes=[
            pltpu.VMEM((_NSLOT, _HID, _CH), jnp.float32),
            pltpu.VMEM((_NSLOT, _BATCH, _CH), jnp.float32),
            pltpu.SemaphoreType.DMA((_NSLOT,)),
            pltpu.SemaphoreType.DMA((_NSLOT, 2)),
        ],
        compiler_params=pltpu.CompilerParams(
            vmem_limit_bytes=120 * 1024 * 1024),
    )(h, b2m, w2)


def _tail1_body(h_ref, w2_ref, b2_ref, m_ref, s_ref, lg_in_ref,
                logits_ref, logz_ref):
    del lg_in_ref
    tile = jnp.dot(h_ref[...], w2_ref[...],
                   preferred_element_type=jnp.float32) + b2_ref[...]
    logits_ref[...] = tile
    col = jax.lax.broadcasted_iota(jnp.int32, (1, _CH), 1)
    tile = jnp.where(col < _TAILW, tile, -jnp.inf)
    tmax = jnp.max(tile, axis=1, keepdims=True)
    m = m_ref[...]
    m_new = jnp.maximum(m, tmax)
    s = (s_ref[...] * jnp.exp(m - m_new)
         + jnp.sum(jnp.exp(tile - m_new), axis=1, keepdims=True))
    logz_ref[...] = m_new + jnp.log(s)


def _tail1(h, w2, b2r, m, s, logits_main):
    return pl.pallas_call(
        _tail1_body,
        grid=(1,),
        in_specs=[
            pl.BlockSpec((_BATCH, _HID), lambda i: (0, 0)),
            pl.BlockSpec((_HID, _CH), lambda i: (0, _NFULL)),
            pl.BlockSpec((1, _CH), lambda i: (0, _NFULL)),
            pl.BlockSpec((_BATCH, 1), lambda i: (0, 0)),
            pl.BlockSpec((_BATCH, 1), lambda i: (0, 0)),
            pl.BlockSpec(memory_space=pltpu.MemorySpace.HBM),
        ],
        out_specs=[
            pl.BlockSpec((_BATCH, _CH), lambda i: (0, _NFULL)),
            pl.BlockSpec((_BATCH, 1), lambda i: (0, 0)),
        ],
        out_shape=[
            jax.ShapeDtypeStruct((_BATCH, _VOCAB), jnp.float32),
            jax.ShapeDtypeStruct((_BATCH, 1), jnp.float32),
        ],
        input_output_aliases={5: 0},
    )(h, w2, b2r, m, s, logits_main)


def _pass2_body(h_ref, b2m_ref, logz_ref, w2_hbm, probas_hbm,
                w2_buf, out_buf, in_sem, out_sem):
    h = h_ref[...]
    logz = logz_ref[...]
    in_cp, out_cp = _chunk_dmas(w2_hbm, probas_hbm, w2_buf, out_buf,
                                in_sem, out_sem)
    for r in range(_NSLOT):
        in_cp(r, r).start()

    def step(k, carry):
        for r in range(_NSLOT):
            j = k * _NSLOT + r
            in_cp(j, r).wait()
            tile = jnp.dot(h, w2_buf[r, :, :],
                           preferred_element_type=jnp.float32)
            tile = tile + (b2m_ref[pl.ds(j, 1), :] - logz)

            @pl.when(k >= 1)
            def _():
                out_cp(j, r, 0).wait()
                out_cp(j, r, 1).wait()

            out_buf[r, :, :] = tile
            out_cp(j, r, 0).start()
            out_cp(j, r, 1).start()

            @pl.when(k < _NFULL // _NSLOT - 1)
            def _():
                in_cp(j + _NSLOT, r).start()
        return carry

    lax.fori_loop(0, _NFULL // _NSLOT, step, 0)
    for r in range(_NSLOT):
        out_cp(_NFULL - _NSLOT + r, r, 0).wait()
        out_cp(_NFULL - _NSLOT + r, r, 1).wait()


def _pass2(h, b2m, logz, w2):
    return pl.pallas_call(
        _pass2_body,
        in_specs=[
            pl.BlockSpec(memory_space=pltpu.MemorySpace.VMEM),
            pl.BlockSpec(memory_space=pltpu.MemorySpace.VMEM),
            pl.BlockSpec(memory_space=pltpu.MemorySpace.VMEM),
            pl.BlockSpec(memory_space=pltpu.MemorySpace.HBM),
        ],
        out_specs=pl.BlockSpec(memory_space=pltpu.MemorySpace.HBM),
        out_shape=jax.ShapeDtypeStruct((_BATCH, _VOCAB), jnp.float32),
        scratch_shapes=[
            pltpu.VMEM((_NSLOT, _HID, _CH), jnp.float32),
            pltpu.VMEM((_NSLOT, _BATCH, _CH), jnp.float32),
            pltpu.SemaphoreType.DMA((_NSLOT,)),
            pltpu.SemaphoreType.DMA((_NSLOT, 2)),
        ],
        compiler_params=pltpu.CompilerParams(
            vmem_limit_bytes=120 * 1024 * 1024),
    )(h, b2m, logz, w2)


def _tail2_body(h_ref, w2_ref, b2_ref, logz_ref, pr_in_ref, probas_ref):
    del pr_in_ref
    tile = jnp.dot(h_ref[...], w2_ref[...],
                   preferred_element_type=jnp.float32)
    probas_ref[...] = tile + b2_ref[...] - logz_ref[...]


def _tail2(h, w2, b2r, logz, probas_main):
    return pl.pallas_call(
        _tail2_body,
        grid=(1,),
        in_specs=[
            pl.BlockSpec((_BATCH, _HID), lambda i: (0, 0)),
            pl.BlockSpec((_HID, _CH), lambda i: (0, _NFULL)),
            pl.BlockSpec((1, _CH), lambda i: (0, _NFULL)),
            pl.BlockSpec((_BATCH, 1), lambda i: (0, 0)),
            pl.BlockSpec(memory_space=pltpu.MemorySpace.HBM),
        ],
        out_specs=pl.BlockSpec((_BATCH, _CH), lambda i: (0, _NFULL)),
        out_shape=jax.ShapeDtypeStruct((_BATCH, _VOCAB), jnp.float32),
        input_output_aliases={4: 0},
    )(h, w2, b2r, logz, probas_main)


def kernel(inputs, embed_table, W1, b1, W2, b2):
    idx = inputs.reshape(-1).astype(jnp.int32)
    x = _sc_gather(embed_table, idx)
    x = x.reshape(_BATCH, _CTX * _EMBED)
    h = _mlp1(x, W1, b1.reshape(1, _HID))
    b2r = b2.reshape(1, _VOCAB)
    b2m = b2[:_MAIN].reshape(_NFULL, _CH)
    logits_main, m, s = _pass1(h, b2m, W2)
    logits, logz = _tail1(h, W2, b2r, m, s, logits_main)
    probas_main = _pass2(h, b2m, logz, W2)
    probas = _tail2(h, W2, b2r, logz, probas_main)
    return (logits, probas)


# speedup vs baseline: 1.0418x; 1.0418x over previous
import functools

import jax
import jax.numpy as jnp
from jax import lax
from jax.experimental import pallas as pl
from jax.experimental.pallas import tpu as pltpu
from jax.experimental.pallas import tpu_sc as plsc

_VOCAB = 100000
_EMBED = 64
_CTX = 20
_HID = 256
_BATCH = 1024

_CH = 2048
_NFULL = _VOCAB // _CH
_MAIN = _NFULL * _CH
_TAILW = _VOCAB - _MAIN
_NSLOT = 4
_HALF = _BATCH // 2


_IDX_CHUNK = 128


def _sc_gather(table, idx):
    info = plsc.get_sparse_core_info()
    nc, ns = info.num_cores, info.num_subcores
    nw = nc * ns
    n = idx.shape[0]
    assert n % (nw * _IDX_CHUNK) == 0
    per_w = n // nw
    chunks = per_w // _IDX_CHUNK
    idx3 = idx.reshape(nw, chunks, _IDX_CHUNK)
    mesh = plsc.VectorSubcoreMesh(core_axis_name="c", subcore_axis_name="s")

    @functools.partial(
        pl.kernel,
        mesh=mesh,
        out_type=jax.ShapeDtypeStruct((n, _EMBED), jnp.float32),
        scratch_types=[
            pltpu.VMEM((chunks, _IDX_CHUNK), jnp.int32),
            pltpu.VMEM((per_w, _EMBED), jnp.float32),
            pltpu.SemaphoreType.DMA,
        ],
        compiler_params=pltpu.CompilerParams(use_tc_tiling_on_sc=False),
    )
    def gather_k(table_hbm, idx_hbm, out_hbm, idx_v, rows_v, sem):
        wid = lax.axis_index("s") * nc + lax.axis_index("c")
        base = wid * per_w
        pltpu.sync_copy(idx_hbm.at[wid], idx_v)
        cps = []
        for i in range(chunks):
            cps.append(pltpu.async_copy(
                table_hbm.at[idx_v.at[i]],
                rows_v.at[pl.ds(i * _IDX_CHUNK, _IDX_CHUNK)],
                sem,
            ))
        for cp in cps:
            cp.wait()
        pltpu.sync_copy(rows_v, out_hbm.at[pl.ds(base, per_w)])

    return gather_k(table, idx3)


def _mlp1_body(x_ref, w1_ref, b1_ref, h_ref):
    acc = jnp.dot(x_ref[...], w1_ref[...],
                  preferred_element_type=jnp.float32,
                  precision=lax.Precision.HIGHEST)
    h_ref[...] = jnp.maximum(acc + b1_ref[...], 0.0)


def _mlp1(x, w1, b1r):
    return pl.pallas_call(
        _mlp1_body,
        out_shape=jax.ShapeDtypeStruct((_BATCH, _HID), jnp.float32),
    )(x, w1, b1r)


def _chunk_dmas(w2_hbm, big_hbm, w2_buf, out_buf, in_sem, out_sem):
    def in_cp(j, slot):
        return pltpu.make_async_copy(
            w2_hbm.at[:, pl.ds(pl.multiple_of(j * _CH, _CH), _CH)],
            w2_buf.at[slot],
            in_sem.at[slot])

    def out_cp(j, slot, half):
        return pltpu.make_async_copy(
            out_buf.at[slot, pl.ds(half * _HALF, _HALF)],
            big_hbm.at[pl.ds(half * _HALF, _HALF),
                       pl.ds(pl.multiple_of(j * _CH, _CH), _CH)],
            out_sem.at[slot, half])

    return in_cp, out_cp


def _pass1_body(h_ref, b2m_ref, w2_hbm, logits_hbm, m_ref, s_ref,
                w2_buf, out_buf, in_sem, out_sem):
    h = h_ref[...]
    in_cp, out_cp = _chunk_dmas(w2_hbm, logits_hbm, w2_buf, out_buf,
                                in_sem, out_sem)
    for r in range(_NSLOT):
        in_cp(r, r).start()

    def step(k, carry):
        m, s = carry
        for r in range(_NSLOT):
            j = k * _NSLOT + r
            in_cp(j, r).wait()
            tile = jnp.dot(h, w2_buf[r, :, :],
                           preferred_element_type=jnp.float32)
            tile = tile + b2m_ref[pl.ds(j, 1), :]

            @pl.when(k >= 1)
            def _():
                out_cp(j, r, 0).wait()
                out_cp(j, r, 1).wait()

            out_buf[r, :, :] = tile
            out_cp(j, r, 0).start()
            out_cp(j, r, 1).start()

            @pl.when(k < _NFULL // _NSLOT - 1)
            def _():
                in_cp(j + _NSLOT, r).start()

            tmax = jnp.max(tile, axis=1, keepdims=True)
            m_new = jnp.maximum(m, tmax)
            s = (s * jnp.exp(m - m_new)
                 + jnp.sum(jnp.exp(tile - m_new), axis=1, keepdims=True))
            m = m_new
        return m, s

    m0 = jnp.full((_BATCH, 1), -jnp.inf, jnp.float32)
    s0 = jnp.zeros((_BATCH, 1), jnp.float32)
    m, s = lax.fori_loop(0, _NFULL // _NSLOT, step, (m0, s0))
    m_ref[...] = m
    s_ref[...] = s
    for r in range(_NSLOT):
        out_cp(_NFULL - _NSLOT + r, r, 0).wait()
        out_cp(_NFULL - _NSLOT + r, r, 1).wait()


def _pass1(h, b2m, w2):
    return pl.pallas_call(
        _pass1_body,
        in_specs=[
            pl.BlockSpec(memory_space=pltpu.MemorySpace.VMEM),
            pl.BlockSpec(memory_space=pltpu.MemorySpace.VMEM),
            pl.BlockSpec(memory_space=pltpu.MemorySpace.HBM),
        ],
        out_specs=[
            pl.BlockSpec(memory_space=pltpu.MemorySpace.HBM),
            pl.BlockSpec(memory_space=pltpu.MemorySpace.VMEM),
            pl.BlockSpec(memory_space=pltpu.MemorySpace.VMEM),
        ],
        out_shape=[
            jax.ShapeDtypeStruct((_BATCH, _VOCAB), jnp.float32),
            jax.ShapeDtypeStruct((_BATCH, 1), jnp.float32),
            jax.ShapeDtypeStruct((_BATCH, 1), jnp.float32),
        ],
        scratch_shapes=[
            pltpu.VMEM((_NSLOT, _HID, _CH), jnp.float32),
            pltpu.VMEM((_NSLOT, _BATCH, _CH), jnp.float32),
            pltpu.SemaphoreType.DMA((_NSLOT,)),
            pltpu.SemaphoreType.DMA((_NSLOT, 2)),
        ],
        compiler_params=pltpu.CompilerParams(
            vmem_limit_bytes=120 * 1024 * 1024),
    )(h, b2m, w2)


def _tail1_body(h_ref, w2_ref, b2_ref, m_ref, s_ref, lg_in_ref,
                logits_ref, logz_ref):
    del lg_in_ref
    tile = jnp.dot(h_ref[...], w2_ref[...],
                   preferred_element_type=jnp.float32) + b2_ref[...]
    logits_ref[...] = tile
    col = jax.lax.broadcasted_iota(jnp.int32, (1, _CH), 1)
    tile = jnp.where(col < _TAILW, tile, -jnp.inf)
    tmax = jnp.max(tile, axis=1, keepdims=True)
    m = m_ref[...]
    m_new = jnp.maximum(m, tmax)
    s = (s_ref[...] * jnp.exp(m - m_new)
         + jnp.sum(jnp.exp(tile - m_new), axis=1, keepdims=True))
    logz_ref[...] = m_new + jnp.log(s)


def _tail1(h, w2, b2r, m, s, logits_main):
    return pl.pallas_call(
        _tail1_body,
        grid=(1,),
        in_specs=[
            pl.BlockSpec((_BATCH, _HID), lambda i: (0, 0)),
            pl.BlockSpec((_HID, _CH), lambda i: (0, _NFULL)),
            pl.BlockSpec((1, _CH), lambda i: (0, _NFULL)),
            pl.BlockSpec((_BATCH, 1), lambda i: (0, 0)),
            pl.BlockSpec((_BATCH, 1), lambda i: (0, 0)),
            pl.BlockSpec(memory_space=pltpu.MemorySpace.HBM),
        ],
        out_specs=[
            pl.BlockSpec((_BATCH, _CH), lambda i: (0, _NFULL)),
            pl.BlockSpec((_BATCH, 1), lambda i: (0, 0)),
        ],
        out_shape=[
            jax.ShapeDtypeStruct((_BATCH, _VOCAB), jnp.float32),
            jax.ShapeDtypeStruct((_BATCH, 1), jnp.float32),
        ],
        input_output_aliases={5: 0},
    )(h, w2, b2r, m, s, logits_main)


def _pass2_body(h_ref, b2m_ref, logz_ref, w2_hbm, probas_hbm,
                w2_buf, out_buf, in_sem, out_sem):
    h = h_ref[...]
    logz = logz_ref[...]
    in_cp, out_cp = _chunk_dmas(w2_hbm, probas_hbm, w2_buf, out_buf,
                                in_sem, out_sem)
    for r in range(_NSLOT):
        in_cp(r, r).start()

    def step(k, carry):
        for r in range(_NSLOT):
            j = k * _NSLOT + r
            in_cp(j, r).wait()
            tile = jnp.dot(h, w2_buf[r, :, :],
                           preferred_element_type=jnp.float32)
            tile = tile + (b2m_ref[pl.ds(j, 1), :] - logz)

            @pl.when(k >= 1)
            def _():
                out_cp(j, r, 0).wait()
                out_cp(j, r, 1).wait()

            out_buf[r, :, :] = tile
            out_cp(j, r, 0).start()
            out_cp(j, r, 1).start()

            @pl.when(k < _NFULL // _NSLOT - 1)
            def _():
                in_cp(j + _NSLOT, r).start()
        return carry

    lax.fori_loop(0, _NFULL // _NSLOT, step, 0)
    for r in range(_NSLOT):
        out_cp(_NFULL - _NSLOT + r, r, 0).wait()
        out_cp(_NFULL - _NSLOT + r, r, 1).wait()


def _pass2(h, b2m, logz, w2):
    return pl.pallas_call(
        _pass2_body,
        in_specs=[
            pl.BlockSpec(memory_space=pltpu.MemorySpace.VMEM),
            pl.BlockSpec(memory_space=pltpu.MemorySpace.VMEM),
            pl.BlockSpec(memory_space=pltpu.MemorySpace.VMEM),
            pl.BlockSpec(memory_space=pltpu.MemorySpace.HBM),
        ],
        out_specs=pl.BlockSpec(memory_space=pltpu.MemorySpace.HBM),
        out_shape=jax.ShapeDtypeStruct((_BATCH, _VOCAB), jnp.float32),
        scratch_shapes=[
            pltpu.VMEM((_NSLOT, _HID, _CH), jnp.float32),
            pltpu.VMEM((_NSLOT, _BATCH, _CH), jnp.float32),
            pltpu.SemaphoreType.DMA((_NSLOT,)),
            pltpu.SemaphoreType.DMA((_NSLOT, 2)),
        ],
        compiler_params=pltpu.CompilerParams(
            vmem_limit_bytes=120 * 1024 * 1024),
    )(h, b2m, logz, w2)


def _tail2_body(h_ref, w2_ref, b2_ref, logz_ref, pr_in_ref, probas_ref):
    del pr_in_ref
    tile = jnp.dot(h_ref[...], w2_ref[...],
                   preferred_element_type=jnp.float32)
    probas_ref[...] = tile + b2_ref[...] - logz_ref[...]


def _tail2(h, w2, b2r, logz, probas_main):
    return pl.pallas_call(
        _tail2_body,
        grid=(1,),
        in_specs=[
            pl.BlockSpec((_BATCH, _HID), lambda i: (0, 0)),
            pl.BlockSpec((_HID, _CH), lambda i: (0, _NFULL)),
            pl.BlockSpec((1, _CH), lambda i: (0, _NFULL)),
            pl.BlockSpec((_BATCH, 1), lambda i: (0, 0)),
            pl.BlockSpec(memory_space=pltpu.MemorySpace.HBM),
        ],
        out_specs=pl.BlockSpec((_BATCH, _CH), lambda i: (0, _NFULL)),
        out_shape=jax.ShapeDtypeStruct((_BATCH, _VOCAB), jnp.float32),
        input_output_aliases={4: 0},
    )(h, w2, b2r, logz, probas_main)


def kernel(inputs, embed_table, W1, b1, W2, b2):
    idx = inputs.reshape(-1).astype(jnp.int32)
    x = _sc_gather(embed_table, idx)
    x = x.reshape(_BATCH, _CTX * _EMBED)
    h = _mlp1(x, W1, b1.reshape(1, _HID))
    b2r = b2.reshape(1, _VOCAB)
    b2m = b2[:_MAIN].reshape(_NFULL, _CH)
    logits_main, m, s = _pass1(h, b2m, W2)
    logits, logz = _tail1(h, W2, b2r, m, s, logits_main)
    probas = logits - logz
    return (logits, probas)


# speedup vs baseline: 2.1209x; 2.0357x over previous
import functools

import jax
import jax.numpy as jnp
from jax import lax
from jax.experimental import pallas as pl
from jax.experimental.pallas import tpu as pltpu
from jax.experimental.pallas import tpu_sc as plsc

_VOCAB = 100000
_EMBED = 64
_CTX = 20
_HID = 256
_BATCH = 1024

_CH = 2048
_NFULL = _VOCAB // _CH
_MAIN = _NFULL * _CH
_TAILW = _VOCAB - _MAIN
_NSLOT = 4
_HROWS = _CH // 2

_TDIMS = (((0,), (0,)), ((), ()))


_IDX_CHUNK = 128


def _sc_gather(table, idx):
    info = plsc.get_sparse_core_info()
    nc, ns = info.num_cores, info.num_subcores
    nw = nc * ns
    n = idx.shape[0]
    assert n % (nw * _IDX_CHUNK) == 0
    per_w = n // nw
    chunks = per_w // _IDX_CHUNK
    idx3 = idx.reshape(nw, chunks, _IDX_CHUNK)
    mesh = plsc.VectorSubcoreMesh(core_axis_name="c", subcore_axis_name="s")

    @functools.partial(
        pl.kernel,
        mesh=mesh,
        out_type=jax.ShapeDtypeStruct((n, _EMBED), jnp.float32),
        scratch_types=[
            pltpu.VMEM((chunks, _IDX_CHUNK), jnp.int32),
            pltpu.VMEM((per_w, _EMBED), jnp.float32),
            pltpu.SemaphoreType.DMA,
        ],
        compiler_params=pltpu.CompilerParams(use_tc_tiling_on_sc=False),
    )
    def gather_k(table_hbm, idx_hbm, out_hbm, idx_v, rows_v, sem):
        wid = lax.axis_index("s") * nc + lax.axis_index("c")
        base = wid * per_w
        pltpu.sync_copy(idx_hbm.at[wid], idx_v)
        cps = []
        for i in range(chunks):
            cps.append(pltpu.async_copy(
                table_hbm.at[idx_v.at[i]],
                rows_v.at[pl.ds(i * _IDX_CHUNK, _IDX_CHUNK)],
                sem,
            ))
        for cp in cps:
            cp.wait()
        pltpu.sync_copy(rows_v, out_hbm.at[pl.ds(base, per_w)])

    return gather_k(table, idx3)


def _mlp1_body(x_ref, w1_ref, b1_ref, h_ref):
    acc = jnp.dot(x_ref[...], w1_ref[...],
                  preferred_element_type=jnp.float32,
                  precision=lax.Precision.HIGHEST)
    h_ref[...] = jnp.maximum(acc + b1_ref[...], 0.0)


def _mlp1(x, w1, b1r):
    return pl.pallas_call(
        _mlp1_body,
        out_shape=jax.ShapeDtypeStruct((_BATCH, _HID), jnp.float32),
    )(x, w1, b1r)


def _chunk_dmas(w2_hbm, big_hbm, w2_buf, out_buf, in_sem, out_sem):
    def in_cp(j, slot):
        return pltpu.make_async_copy(
            w2_hbm.at[:, pl.ds(pl.multiple_of(j * _CH, _CH), _CH)],
            w2_buf.at[slot],
            in_sem.at[slot])

    def out_cp(j, slot, half):
        base = pl.multiple_of(j * _CH + half * _HROWS, _HROWS)
        return pltpu.make_async_copy(
            out_buf.at[slot, pl.ds(half * _HROWS, _HROWS)],
            big_hbm.at[pl.ds(base, _HROWS)],
            out_sem.at[slot, half])

    return in_cp, out_cp


def _pass1_body(ht_ref, b2m_ref, w2_hbm, logits_hbm, m_ref, s_ref,
                w2_buf, out_buf, in_sem, out_sem):
    ht = ht_ref[...]
    in_cp, out_cp = _chunk_dmas(w2_hbm, logits_hbm, w2_buf, out_buf,
                                in_sem, out_sem)
    for r in range(_NSLOT):
        in_cp(r, r).start()

    def step(k, carry):
        m, s = carry
        for r in range(_NSLOT):
            j = k * _NSLOT + r
            in_cp(j, r).wait()
            tile = lax.dot_general(w2_buf[r, :, :], ht, _TDIMS,
                                   preferred_element_type=jnp.float32)
            b2c = jnp.transpose(b2m_ref[pl.ds(j, 1), :])
            tile = tile + b2c

            @pl.when(k >= 1)
            def _():
                out_cp(j, r, 0).wait()
                out_cp(j, r, 1).wait()

            out_buf[r, :, :] = tile
            out_cp(j, r, 0).start()
            out_cp(j, r, 1).start()

            @pl.when(k < _NFULL // _NSLOT - 1)
            def _():
                in_cp(j + _NSLOT, r).start()

            tmax = jnp.max(tile, axis=0, keepdims=True)
            m_new = jnp.maximum(m, tmax)
            s = (s * jnp.exp(m - m_new)
                 + jnp.sum(jnp.exp(tile - m_new), axis=0, keepdims=True))
            m = m_new
        return m, s

    m0 = jnp.full((1, _BATCH), -jnp.inf, jnp.float32)
    s0 = jnp.zeros((1, _BATCH), jnp.float32)
    m, s = lax.fori_loop(0, _NFULL // _NSLOT, step, (m0, s0))
    m_ref[...] = m
    s_ref[...] = s
    for r in range(_NSLOT):
        out_cp(_NFULL - _NSLOT + r, r, 0).wait()
        out_cp(_NFULL - _NSLOT + r, r, 1).wait()


def _pass1(ht, b2m, w2):
    return pl.pallas_call(
        _pass1_body,
        in_specs=[
            pl.BlockSpec(memory_space=pltpu.MemorySpace.VMEM),
            pl.BlockSpec(memory_space=pltpu.MemorySpace.VMEM),
            pl.BlockSpec(memory_space=pltpu.MemorySpace.HBM),
        ],
        out_specs=[
            pl.BlockSpec(memory_space=pltpu.MemorySpace.HBM),
            pl.BlockSpec(memory_space=pltpu.MemorySpace.VMEM),
            pl.BlockSpec(memory_space=pltpu.MemorySpace.VMEM),
        ],
        out_shape=[
            jax.ShapeDtypeStruct((_VOCAB, _BATCH), jnp.float32),
            jax.ShapeDtypeStruct((1, _BATCH), jnp.float32),
            jax.ShapeDtypeStruct((1, _BATCH), jnp.float32),
        ],
        scratch_shapes=[
            pltpu.VMEM((_NSLOT, _HID, _CH), jnp.float32),
            pltpu.VMEM((_NSLOT, _CH, _BATCH), jnp.float32),
            pltpu.SemaphoreType.DMA((_NSLOT,)),
            pltpu.SemaphoreType.DMA((_NSLOT, 2)),
        ],
        compiler_params=pltpu.CompilerParams(
            vmem_limit_bytes=120 * 1024 * 1024),
    )(ht, b2m, w2)


def _tail1_body(ht_ref, w2_ref, b2t_ref, m_ref, s_ref, lg_in_ref,
                logits_ref, logz_ref):
    del lg_in_ref
    tile = lax.dot_general(w2_ref[...], ht_ref[...], _TDIMS,
                           preferred_element_type=jnp.float32)
    tile = tile + b2t_ref[...]
    logits_ref[...] = tile
    row = jax.lax.broadcasted_iota(jnp.int32, (_CH, 1), 0)
    tile = jnp.where(row < _TAILW, tile, -jnp.inf)
    tmax = jnp.max(tile, axis=0, keepdims=True)
    m = m_ref[...]
    m_new = jnp.maximum(m, tmax)
    s = (s_ref[...] * jnp.exp(m - m_new)
         + jnp.sum(jnp.exp(tile - m_new), axis=0, keepdims=True))
    logz_ref[...] = m_new + jnp.log(s)


def _tail1(ht, w2, b2t, m, s, logits_main):
    return pl.pallas_call(
        _tail1_body,
        grid=(1,),
        in_specs=[
            pl.BlockSpec((_HID, _BATCH), lambda i: (0, 0)),
            pl.BlockSpec((_HID, _CH), lambda i: (0, _NFULL)),
            pl.BlockSpec((_CH, 1), lambda i: (0, 0)),
            pl.BlockSpec((1, _BATCH), lambda i: (0, 0)),
            pl.BlockSpec((1, _BATCH), lambda i: (0, 0)),
            pl.BlockSpec(memory_space=pltpu.MemorySpace.HBM),
        ],
        out_specs=[
            pl.BlockSpec((_CH, _BATCH), lambda i: (_NFULL, 0)),
            pl.BlockSpec((1, _BATCH), lambda i: (0, 0)),
        ],
        out_shape=[
            jax.ShapeDtypeStruct((_VOCAB, _BATCH), jnp.float32),
            jax.ShapeDtypeStruct((1, _BATCH), jnp.float32),
        ],
        input_output_aliases={5: 0},
    )(ht, w2, b2t, m, s, logits_main)


def _pass2_body(ht_ref, b2m_ref, logz_ref, w2_hbm, probas_hbm,
                w2_buf, out_buf, in_sem, out_sem):
    ht = ht_ref[...]
    logz = logz_ref[...]
    in_cp, out_cp = _chunk_dmas(w2_hbm, probas_hbm, w2_buf, out_buf,
                                in_sem, out_sem)
    for r in range(_NSLOT):
        in_cp(r, r).start()

    def step(k, carry):
        for r in range(_NSLOT):
            j = k * _NSLOT + r
            in_cp(j, r).wait()
            tile = lax.dot_general(w2_buf[r, :, :], ht, _TDIMS,
                                   preferred_element_type=jnp.float32)
            b2c = jnp.transpose(b2m_ref[pl.ds(j, 1), :])
            tile = (tile + b2c) - logz

            @pl.when(k >= 1)
            def _():
                out_cp(j, r, 0).wait()
                out_cp(j, r, 1).wait()

            out_buf[r, :, :] = tile
            out_cp(j, r, 0).start()
            out_cp(j, r, 1).start()

            @pl.when(k < _NFULL // _NSLOT - 1)
            def _():
                in_cp(j + _NSLOT, r).start()
        return carry

    lax.fori_loop(0, _NFULL // _NSLOT, step, 0)
    for r in range(_NSLOT):
        out_cp(_NFULL - _NSLOT + r, r, 0).wait()
        out_cp(_NFULL - _NSLOT + r, r, 1).wait()


def _pass2(ht, b2m, logz, w2):
    return pl.pallas_call(
        _pass2_body,
        in_specs=[
            pl.BlockSpec(memory_space=pltpu.MemorySpace.VMEM),
            pl.BlockSpec(memory_space=pltpu.MemorySpace.VMEM),
            pl.BlockSpec(memory_space=pltpu.MemorySpace.VMEM),
            pl.BlockSpec(memory_space=pltpu.MemorySpace.HBM),
        ],
        out_specs=pl.BlockSpec(memory_space=pltpu.MemorySpace.HBM),
        out_shape=jax.ShapeDtypeStruct((_VOCAB, _BATCH), jnp.float32),
        scratch_shapes=[
            pltpu.VMEM((_NSLOT, _HID, _CH), jnp.float32),
            pltpu.VMEM((_NSLOT, _CH, _BATCH), jnp.float32),
            pltpu.SemaphoreType.DMA((_NSLOT,)),
            pltpu.SemaphoreType.DMA((_NSLOT, 2)),
        ],
        compiler_params=pltpu.CompilerParams(
            vmem_limit_bytes=120 * 1024 * 1024),
    )(ht, b2m, logz, w2)


def _tail2_body(ht_ref, w2_ref, b2t_ref, logz_ref, pr_in_ref, probas_ref):
    del pr_in_ref
    tile = lax.dot_general(w2_ref[...], ht_ref[...], _TDIMS,
                           preferred_element_type=jnp.float32)
    probas_ref[...] = (tile + b2t_ref[...]) - logz_ref[...]


def _tail2(ht, w2, b2t, logz, probas_main):
    return pl.pallas_call(
        _tail2_body,
        grid=(1,),
        in_specs=[
            pl.BlockSpec((_HID, _BATCH), lambda i: (0, 0)),
            pl.BlockSpec((_HID, _CH), lambda i: (0, _NFULL)),
            pl.BlockSpec((_CH, 1), lambda i: (0, 0)),
            pl.BlockSpec((1, _BATCH), lambda i: (0, 0)),
            pl.BlockSpec(memory_space=pltpu.MemorySpace.HBM),
        ],
        out_specs=pl.BlockSpec((_CH, _BATCH), lambda i: (_NFULL, 0)),
        out_shape=jax.ShapeDtypeStruct((_VOCAB, _BATCH), jnp.float32),
        input_output_aliases={4: 0},
    )(ht, w2, b2t, logz, probas_main)


def kernel(inputs, embed_table, W1, b1, W2, b2):
    idx = inputs.reshape(-1).astype(jnp.int32)
    x = _sc_gather(embed_table, idx)
    x = x.reshape(_BATCH, _CTX * _EMBED)
    h = _mlp1(x, W1, b1.reshape(1, _HID))
    ht = h.T
    b2m = b2[:_MAIN].reshape(_NFULL, _CH)
    b2t = jnp.pad(b2[_MAIN:], (0, _CH - _TAILW)).reshape(_CH, 1)
    logitsT_main, m, s = _pass1(ht, b2m, W2)
    logitsT, logz = _tail1(ht, W2, b2t, m, s, logitsT_main)
    probasT_main = _pass2(ht, b2m, logz, W2)
    probasT = _tail2(ht, W2, b2t, logz, probasT_main)
    return (logitsT.T, probasT.T)


# speedup vs baseline: 2.2305x; 1.0517x over previous
import functools

import jax
import jax.numpy as jnp
from jax import lax
from jax.experimental import pallas as pl
from jax.experimental.pallas import tpu as pltpu
from jax.experimental.pallas import tpu_sc as plsc

_VOCAB = 100000
_EMBED = 64
_CTX = 20
_HID = 256
_BATCH = 1024

_CH = 2048
_NFULL = _VOCAB // _CH
_MAIN = _NFULL * _CH
_TAILW = _VOCAB - _MAIN
_NSLOT = 3
_HROWS = _CH // 2


_IDX_CHUNK = 128


def _sc_gather(table, idx):
    info = plsc.get_sparse_core_info()
    nc, ns = info.num_cores, info.num_subcores
    nw = nc * ns
    n = idx.shape[0]
    assert n % (nw * _IDX_CHUNK) == 0
    per_w = n // nw
    chunks = per_w // _IDX_CHUNK
    idx3 = idx.reshape(nw, chunks, _IDX_CHUNK)
    mesh = plsc.VectorSubcoreMesh(core_axis_name="c", subcore_axis_name="s")

    @functools.partial(
        pl.kernel,
        mesh=mesh,
        out_type=jax.ShapeDtypeStruct((n, _EMBED), jnp.float32),
        scratch_types=[
            pltpu.VMEM((chunks, _IDX_CHUNK), jnp.int32),
            pltpu.VMEM((per_w, _EMBED), jnp.float32),
            pltpu.SemaphoreType.DMA,
        ],
        compiler_params=pltpu.CompilerParams(use_tc_tiling_on_sc=False),
    )
    def gather_k(table_hbm, idx_hbm, out_hbm, idx_v, rows_v, sem):
        wid = lax.axis_index("s") * nc + lax.axis_index("c")
        base = wid * per_w
        pltpu.sync_copy(idx_hbm.at[wid], idx_v)
        cps = []
        for i in range(chunks):
            cps.append(pltpu.async_copy(
                table_hbm.at[idx_v.at[i]],
                rows_v.at[pl.ds(i * _IDX_CHUNK, _IDX_CHUNK)],
                sem,
            ))
        for cp in cps:
            cp.wait()
        pltpu.sync_copy(rows_v, out_hbm.at[pl.ds(base, per_w)])

    return gather_k(table, idx3)


_TDIMS = (((0,), (0,)), ((), ()))
_TDIMS_X = (((0,), (1,)), ((), ()))


def _mega_body(x_ref, w1_ref, b1c_ref, b2m_ref, b2t_ref, w2t_ref, w2_hbm,
               logits_hbm, probas_hbm,
               w2_buf, out_buf, tail_buf, in_sem, out_sem, tail_sem):
    ht = jnp.maximum(
        lax.dot_general(w1_ref[...], x_ref[...], _TDIMS_X,
                        preferred_element_type=jnp.float32)
        + b1c_ref[...], 0.0)

    def in_cp(j, slot):
        return pltpu.make_async_copy(
            w2_hbm.at[:, pl.ds(pl.multiple_of(j * _CH, _CH), _CH)],
            w2_buf.at[slot],
            in_sem.at[slot])

    def out_cp(hbm, j, slot, half):
        return pltpu.make_async_copy(
            out_buf.at[slot, pl.ds(half * _HROWS, _HROWS)],
            hbm.at[pl.ds(j * _CH + half * _HROWS, _HROWS)],
            out_sem.at[slot, half])

    def tail_cp(hbm, half):
        return pltpu.make_async_copy(
            tail_buf.at[pl.ds(half * (_TAILW // 2), _TAILW // 2)],
            hbm.at[pl.ds(_MAIN + half * (_TAILW // 2), _TAILW // 2)],
            tail_sem.at[half])

    def sweep(out_hbm, stats, fin, carry0):
        for r in range(_NSLOT):
            in_cp(r, r).start()
        ttile = lax.dot_general(w2t_ref[...], ht, _TDIMS,
                                preferred_element_type=jnp.float32)
        ttile = fin(ttile + b2t_ref[...])
        tail_buf[...] = ttile
        tail_cp(out_hbm, 0).start()
        tail_cp(out_hbm, 1).start()
        if stats:
            m0 = jnp.max(ttile, axis=0, keepdims=True)
            s0 = jnp.sum(jnp.exp(ttile - m0), axis=0, keepdims=True)
            carry0 = (m0, s0)

        def step(k, carry):
            for r in range(_NSLOT):
                j = k * _NSLOT + r
                in_cp(j, r).wait()
                tile = lax.dot_general(w2_buf[r, :, :], ht, _TDIMS,
                                       preferred_element_type=jnp.float32)
                tile = fin(tile + jnp.transpose(b2m_ref[pl.ds(j, 1), :]))

                @pl.when(k >= 1)
                def _():
                    out_cp(out_hbm, j, r, 0).wait()
                    out_cp(out_hbm, j, r, 1).wait()

                out_buf[r, :, :] = tile
                out_cp(out_hbm, j, r, 0).start()
                out_cp(out_hbm, j, r, 1).start()

                @pl.when(k < _NFULL // _NSLOT - 1)
                def _():
                    in_cp(j + _NSLOT, r).start()

                if stats:
                    m, s = carry
                    tmax = jnp.max(tile, axis=0, keepdims=True)
                    m_new = jnp.maximum(m, tmax)
                    s = (s * jnp.exp(m - m_new)
                         + jnp.sum(jnp.exp(tile - m_new), axis=0,
                                   keepdims=True))
                    carry = (m_new, s)
            return carry

        carry = lax.fori_loop(0, _NFULL // _NSLOT, step, carry0)
        for r in range(_NSLOT):
            out_cp(out_hbm, _NFULL - _NSLOT + r, r, 0).wait()
            out_cp(out_hbm, _NFULL - _NSLOT + r, r, 1).wait()
        tail_cp(out_hbm, 0).wait()
        tail_cp(out_hbm, 1).wait()
        return carry

    m, s = sweep(logits_hbm, True, lambda t: t, None)
    logz = m + jnp.log(s)
    sweep(probas_hbm, False, lambda t: t - logz, 0)


def _mega(x, w1, b1c, b2m, b2t, w2t, w2):
    return pl.pallas_call(
        _mega_body,
        in_specs=[
            pl.BlockSpec(memory_space=pltpu.MemorySpace.VMEM),
            pl.BlockSpec(memory_space=pltpu.MemorySpace.VMEM),
            pl.BlockSpec(memory_space=pltpu.MemorySpace.VMEM),
            pl.BlockSpec(memory_space=pltpu.MemorySpace.VMEM),
            pl.BlockSpec(memory_space=pltpu.MemorySpace.VMEM),
            pl.BlockSpec(memory_space=pltpu.MemorySpace.VMEM),
            pl.BlockSpec(memory_space=pltpu.MemorySpace.HBM),
        ],
        out_specs=[
            pl.BlockSpec(memory_space=pltpu.MemorySpace.HBM),
            pl.BlockSpec(memory_space=pltpu.MemorySpace.HBM),
        ],
        out_shape=[
            jax.ShapeDtypeStruct((_VOCAB, _BATCH), jnp.float32),
            jax.ShapeDtypeStruct((_VOCAB, _BATCH), jnp.float32),
        ],
        scratch_shapes=[
            pltpu.VMEM((_NSLOT, _HID, _CH), jnp.float32),
            pltpu.VMEM((_NSLOT, _CH, _BATCH), jnp.float32),
            pltpu.VMEM((_TAILW, _BATCH), jnp.float32),
            pltpu.SemaphoreType.DMA((_NSLOT,)),
            pltpu.SemaphoreType.DMA((_NSLOT, 2)),
            pltpu.SemaphoreType.DMA((2,)),
        ],
        compiler_params=pltpu.CompilerParams(
            vmem_limit_bytes=63 * 1024 * 1024),
    )(x, w1, b1c, b2m, b2t, w2t, w2)


def kernel(inputs, embed_table, W1, b1, W2, b2):
    idx = inputs.reshape(-1).astype(jnp.int32)
    x = _sc_gather(embed_table, idx)
    x = x.reshape(_BATCH, _CTX * _EMBED)
    b1c = b1.reshape(_HID, 1)
    b2m = b2[:_MAIN].reshape(_NFULL, _CH)
    b2t = b2[_MAIN:].reshape(_TAILW, 1)
    w2t = W2[:, _MAIN:]
    logitsT, probasT = _mega(x, W1, b1c, b2m, b2t, w2t, W2)
    return (logitsT.T, probasT.T)
